# baseline (speedup 1.0000x reference)
import functools

import jax
import jax.numpy as jnp
from jax import lax
from jax.experimental import pallas as pl
from jax.experimental.pallas import tpu as pltpu
from jax.experimental.pallas import tpu_sc as plsc

_NUM_GENES = 20000
_B = 64
_V = 512
_D = 64
_PAIRS = _B * _V


def _make_sc_dots():
    info = plsc.get_sparse_core_info()
    nw = info.num_cores * info.num_subcores
    per_w = _PAIRS // nw
    chunk = per_w // 4
    rows_per_w = per_w // _V
    mesh = plsc.VectorSubcoreMesh(core_axis_name="c", subcore_axis_name="s")

    @functools.partial(
        pl.kernel,
        mesh=mesh,
        out_type=jax.ShapeDtypeStruct((_B, _V), jnp.float32),
        scratch_types=[
            pltpu.VMEM((rows_per_w, _V), jnp.int32),
            pltpu.VMEM((rows_per_w, _V, _D), jnp.float32),
            pltpu.VMEM((chunk, _D), jnp.float32),
            pltpu.VMEM((chunk, _D), jnp.float32),
            pltpu.VMEM((rows_per_w, _V), jnp.float32),
            pltpu.SemaphoreType.DMA,
            pltpu.SemaphoreType.DMA,
        ],
        compiler_params=pltpu.CompilerParams(use_tc_tiling_on_sc=False),
    )
    def sc_dots(gid_hbm, feat_hbm, table_hbm, out_hbm,
                idx_v, feat_v, rows0_v, rows1_v, dots_v, sem0, sem1):
        wid = lax.axis_index("s") * info.num_cores + lax.axis_index("c")
        r0 = wid * rows_per_w
        pltpu.sync_copy(gid_hbm.at[pl.ds(r0, rows_per_w)], idx_v)

        for rr in range(rows_per_w):
            def clamp_body(i, _, rr=rr):
                g = idx_v[rr, pl.ds(i * 16, 16)]
                idx_v[rr, pl.ds(i * 16, 16)] = jnp.clip(g, 0, _NUM_GENES - 1)
                return 0
            lax.fori_loop(0, _V // 16, clamp_body, 0)

        rows_bufs = (rows0_v, rows1_v)
        sems = (sem0, sem1)
        chunks_per_row = _V // chunk

        def fire(c):
            rr, cb = c // chunks_per_row, (c % chunks_per_row) * chunk
            return pltpu.async_copy(
                table_hbm.at[idx_v.at[rr, pl.ds(cb, chunk)]],
                rows_bufs[c % 2], sems[c % 2])

        cps = {0: fire(0), 1: fire(1)}
        pltpu.sync_copy(feat_hbm.at[pl.ds(r0, rows_per_w)], feat_v)

        lane = lax.iota(jnp.int32, 16)

        def xl_gather(x, idx):
            return lax.gather(
                x, idx[:, None],
                dimension_numbers=lax.GatherDimensionNumbers(
                    offset_dims=(), collapsed_slice_dims=(0,),
                    start_index_map=(0,)),
                slice_sizes=(1,),
                mode=lax.GatherScatterMode.PROMISE_IN_BOUNDS)

        bfly = [lane ^ 8, lane ^ 4, lane ^ 2, lane ^ 1]

        for c in range(per_w // chunk):
            cps[c].wait()
            rows_v = rows_bufs[c % 2]
            rr, cb = c // chunks_per_row, (c % chunks_per_row) * chunk

            def group_body(grp, _, rr=rr, cb=cb, rows_v=rows_v):
                dots16 = jnp.zeros((16,), jnp.float32)
                for j in range(16):
                    p = grp * 16 + j
                    acc = jnp.zeros((16,), jnp.float32)
                    for k in range(_D // 16):
                        w = rows_v[p, pl.ds(k * 16, 16)]
                        f = feat_v[rr, cb + p, pl.ds(k * 16, 16)]
                        acc = acc + w * f
                    for idx in bfly:
                        acc = acc + xl_gather(acc, idx)
                    dots16 = jnp.where(lane == j, acc, dots16)
                dots_v[rr, pl.ds(cb + grp * 16, 16)] = dots16
                return 0

            lax.fori_loop(0, chunk // 16, group_body, 0)
            if c + 2 < per_w // chunk:
                cps[c + 2] = fire(c + 2)

        pltpu.sync_copy(dots_v, out_hbm.at[pl.ds(r0, rows_per_w)])

    return sc_dots


def _tc_winner_body(gr_ref, gc_ref, mr_ref, mc_ref, dots_ref, b_ref, out_ref):
    V = _V
    grow = gr_ref[0]
    gcol = gc_ref[0]
    mrow = mr_ref[0]
    mcol = mc_ref[0]
    v_lane = lax.broadcasted_iota(jnp.int32, (1, V), 1)
    j_sub = lax.broadcasted_iota(jnp.int32, (V, 1), 0)
    valid_row = (mrow > 0) & (grow < _NUM_GENES)
    valid_col = (mcol > 0) & (gcol < _NUM_GENES)
    krow = jnp.where(valid_row, grow, 20000 + v_lane) << 10
    kcol = jnp.where(valid_col, gcol, 20000 + j_sub) << 10
    cmat = (kcol ^ krow) | (1023 - j_sub)
    cmin = jnp.min(cmat, axis=0, keepdims=True)
    winner = valid_row & (cmin >= 1023 - v_lane)
    contrib = jnp.sum(jnp.where(winner, dots_ref[0], 0.0))
    out_ref[0] = jnp.full((1, 128), contrib + b_ref[0, 0], jnp.float32)


def kernel(features, positions, gene_ids, mask, W, b):
    del positions
    B, V, D = features.shape
    table = W.reshape(_NUM_GENES, _D)

    dots = _make_sc_dots()(gene_ids, features, table)

    gid_row = gene_ids.reshape(B, 1, V)
    gid_col = gene_ids.reshape(B, V, 1)
    msk_row = mask.reshape(B, 1, V)
    msk_col = mask.reshape(B, V, 1)

    out = pl.pallas_call(
        _tc_winner_body,
        grid=(B,),
        in_specs=[
            pl.BlockSpec((1, 1, V), lambda i: (i, 0, 0)),
            pl.BlockSpec((1, V, 1), lambda i: (i, 0, 0)),
            pl.BlockSpec((1, 1, V), lambda i: (i, 0, 0)),
            pl.BlockSpec((1, V, 1), lambda i: (i, 0, 0)),
            pl.BlockSpec((1, 1, V), lambda i: (i, 0, 0)),
            pl.BlockSpec((1, 1), lambda i: (0, 0)),
        ],
        out_specs=pl.BlockSpec((1, 1, 128), lambda i: (i, 0, 0)),
        out_shape=jax.ShapeDtypeStruct((B, 1, 128), jnp.float32),
    )(gid_row, gid_col, msk_row, msk_col, dots.reshape(B, 1, V),
      b.reshape(1, 1))
    return out[:, 0, :1]


# speedup vs baseline: 1.3253x; 1.3253x over previous
import functools

import jax
import jax.numpy as jnp
from jax import lax
from jax.experimental import pallas as pl
from jax.experimental.pallas import tpu as pltpu
from jax.experimental.pallas import tpu_sc as plsc

_NUM_GENES = 20000
_B = 64
_V = 512
_D = 64
_PAIRS = _B * _V


def _make_sc_dots():
    info = plsc.get_sparse_core_info()
    nw = info.num_cores * info.num_subcores
    per_w = _PAIRS // nw
    chunk = per_w // 4
    rows_per_w = per_w // _V
    mesh = plsc.VectorSubcoreMesh(core_axis_name="c", subcore_axis_name="s")

    @functools.partial(
        pl.kernel,
        mesh=mesh,
        out_type=jax.ShapeDtypeStruct((_B, _V), jnp.float32),
        scratch_types=[
            pltpu.VMEM((rows_per_w, _V), jnp.int32),
            pltpu.VMEM((rows_per_w, _V, _D), jnp.float32),
            pltpu.VMEM((chunk, _D), jnp.float32),
            pltpu.VMEM((chunk, _D), jnp.float32),
            pltpu.VMEM((rows_per_w, _V), jnp.float32),
            pltpu.SemaphoreType.DMA,
            pltpu.SemaphoreType.DMA,
        ],
        compiler_params=pltpu.CompilerParams(use_tc_tiling_on_sc=False),
    )
    def sc_dots(gid_hbm, feat_hbm, table_hbm, out_hbm,
                idx_v, feat_v, rows0_v, rows1_v, dots_v, sem0, sem1):
        wid = lax.axis_index("s") * info.num_cores + lax.axis_index("c")
        r0 = wid * rows_per_w
        pltpu.sync_copy(gid_hbm.at[pl.ds(r0, rows_per_w)], idx_v)

        for rr in range(rows_per_w):
            def clamp_body(i, _, rr=rr):
                g = idx_v[rr, pl.ds(i * 16, 16)]
                idx_v[rr, pl.ds(i * 16, 16)] = jnp.clip(g, 0, _NUM_GENES - 1)
                return 0
            lax.fori_loop(0, _V // 16, clamp_body, 0)

        rows_bufs = (rows0_v, rows1_v)
        sems = (sem0, sem1)
        chunks_per_row = _V // chunk

        def fire(c):
            rr, cb = c // chunks_per_row, (c % chunks_per_row) * chunk
            return pltpu.async_copy(
                table_hbm.at[idx_v.at[rr, pl.ds(cb, chunk)]],
                rows_bufs[c % 2], sems[c % 2])

        cps = {0: fire(0), 1: fire(1)}
        pltpu.sync_copy(feat_hbm.at[pl.ds(r0, rows_per_w)], feat_v)

        lane = lax.iota(jnp.int32, 16)

        def xl_gather(x, idx):
            return lax.gather(
                x, idx[:, None],
                dimension_numbers=lax.GatherDimensionNumbers(
                    offset_dims=(), collapsed_slice_dims=(0,),
                    start_index_map=(0,)),
                slice_sizes=(1,),
                mode=lax.GatherScatterMode.PROMISE_IN_BOUNDS)

        bfly = [lane ^ 8, lane ^ 4, lane ^ 2, lane ^ 1]

        for c in range(per_w // chunk):
            cps[c].wait()
            rows_v = rows_bufs[c % 2]
            rr, cb = c // chunks_per_row, (c % chunks_per_row) * chunk

            def group_body(grp, _, rr=rr, cb=cb, rows_v=rows_v):
                dots16 = jnp.zeros((16,), jnp.float32)
                for j in range(16):
                    p = grp * 16 + j
                    acc = jnp.zeros((16,), jnp.float32)
                    for k in range(_D // 16):
                        w = rows_v[p, pl.ds(k * 16, 16)]
                        f = feat_v[rr, cb + p, pl.ds(k * 16, 16)]
                        acc = acc + w * f
                    for idx in bfly:
                        acc = acc + xl_gather(acc, idx)
                    dots16 = jnp.where(lane == j, acc, dots16)
                dots_v[rr, pl.ds(cb + grp * 16, 16)] = dots16
                return 0

            lax.fori_loop(0, chunk // 16, group_body, 0)
            if c + 2 < per_w // chunk:
                cps[c + 2] = fire(c + 2)

        pltpu.sync_copy(dots_v, out_hbm.at[pl.ds(r0, rows_per_w)])

    return sc_dots


def _tc_winner_body(gr_ref, mr_ref, dots_ref, b_ref, out_ref):
    V = _V
    v_lane = lax.broadcasted_iota(jnp.int32, (1, V), 1)
    j_sub = lax.broadcasted_iota(jnp.int32, (V, 1), 0)
    for r in range(8):
        grow = gr_ref[pl.ds(r, 1), :]
        mrow = mr_ref[pl.ds(r, 1), :]
        gcol = jnp.transpose(grow)
        mcol = jnp.transpose(mrow)
        valid_row = (mrow > 0) & (grow < _NUM_GENES)
        valid_col = (mcol > 0) & (gcol < _NUM_GENES)
        krow = jnp.where(valid_row, grow, 20000 + v_lane) << 10
        kcol = jnp.where(valid_col, gcol, 20000 + j_sub) << 10
        cmat = (kcol ^ krow) | (1023 - j_sub)
        cmin = jnp.min(cmat, axis=0, keepdims=True)
        winner = valid_row & (cmin >= 1023 - v_lane)
        drow = dots_ref[pl.ds(r, 1), :]
        contrib = jnp.sum(jnp.where(winner, drow, 0.0))
        out_ref[pl.ds(r, 1), :] = jnp.full((1, 128), contrib + b_ref[0, 0],
                                           jnp.float32)


def kernel(features, positions, gene_ids, mask, W, b):
    del positions
    B, V, D = features.shape
    table = W.reshape(_NUM_GENES, _D)

    dots = _make_sc_dots()(gene_ids, features, table)

    out = pl.pallas_call(
        _tc_winner_body,
        grid=(B // 8,),
        in_specs=[
            pl.BlockSpec((8, V), lambda i: (i, 0)),
            pl.BlockSpec((8, V), lambda i: (i, 0)),
            pl.BlockSpec((8, V), lambda i: (i, 0)),
            pl.BlockSpec((1, 1), lambda i: (0, 0)),
        ],
        out_specs=pl.BlockSpec((8, 128), lambda i: (i, 0)),
        out_shape=jax.ShapeDtypeStruct((B, 128), jnp.float32),
    )(gene_ids, mask, dots, b.reshape(1, 1))
    return out[:, :1]
